# baseline (speedup 1.0000x reference)
import functools

import jax
import jax.numpy as jnp
from jax import lax
from jax.experimental import pallas as pl
from jax.experimental.pallas import tpu as pltpu
from jax.experimental.pallas import tpu_sc as plsc

N = 10000
E = 320000
D = 128
L = 3
XH = 128
CH1 = 128
CH2 = 64

NC = 2
NS = 16
NW = NC * NS
KC = 128
NCH = 79
EPTP = NCH * KC
EPAD = NW * EPTP
HC = KC // 2
RPT = N // NS
NPAD = 10240
RZT = NPAD // NS
DW = 128


@functools.cache
def _sc_kernels():
    mesh = plsc.VectorSubcoreMesh(core_axis_name="c", subcore_axis_name="s",
                                  num_cores=NC, num_subcores=NS)

    @functools.partial(
        pl.kernel,
        out_type=jax.ShapeDtypeStruct((NC * NPAD, DW), jnp.float32),
        mesh=mesh,
        scratch_types=[
            pltpu.VMEM((NCH, KC), jnp.int32),
            pltpu.VMEM((KC, DW), jnp.float32),
            pltpu.VMEM_SHARED((NPAD, DW), jnp.float32),
        ],
    )
    def deg_kernel(dst_hbm, zeros_hbm, ones_hbm, out_hbm, dst_v, ones_v,
                   deg_sh):
        c = lax.axis_index("c")
        s = lax.axis_index("s")
        wid = c * NS + s
        pltpu.sync_copy(zeros_hbm, deg_sh.at[pl.ds(s * RZT, RZT)])
        pltpu.sync_copy(dst_hbm.at[wid], dst_v)
        pltpu.sync_copy(ones_hbm, ones_v)
        plsc.subcore_barrier()

        def body(j, carry):
            pltpu.sync_copy(ones_v, deg_sh.at[dst_v.at[j]], add=True)
            return carry

        lax.fori_loop(0, NCH, body, 0)
        plsc.subcore_barrier()
        pltpu.sync_copy(deg_sh.at[pl.ds(s * RZT, RZT)],
                        out_hbm.at[pl.ds(c * NPAD + s * RZT, RZT)])

    @functools.partial(
        pl.kernel,
        out_type=jax.ShapeDtypeStruct((NC * NPAD, D), jnp.float32),
        mesh=mesh,
        scratch_types=[
            pltpu.VMEM((NCH, KC), jnp.int32),
            pltpu.VMEM((NCH, KC), jnp.int32),
            pltpu.VMEM((KC, D), jnp.float32),
            pltpu.VMEM_SHARED((NPAD, D), jnp.float32),
            pltpu.SemaphoreType.DMA,
        ],
    )
    def agg_kernel(x_hbm, src_hbm, dst_hbm, zrows_hbm, out_hbm,
                   src_v, dst_v, rows_v, acc_sh, sem):
        c = lax.axis_index("c")
        s = lax.axis_index("s")
        wid = c * NS + s
        pltpu.sync_copy(zrows_hbm, acc_sh.at[pl.ds(s * RZT, RZT)])
        pltpu.sync_copy(src_hbm.at[wid], src_v)
        pltpu.sync_copy(dst_hbm.at[wid], dst_v)
        plsc.subcore_barrier()

        def gather(j, h):
            pltpu.async_copy(x_hbm.at[src_v.at[j, pl.ds(h * HC, HC)]],
                             rows_v.at[pl.ds(h * HC, HC)], sem)

        def gwait(j, h):
            pltpu.make_async_copy(x_hbm.at[src_v.at[j, pl.ds(h * HC, HC)]],
                                  rows_v.at[pl.ds(h * HC, HC)], sem).wait()

        def scat(j, h):
            pltpu.sync_copy(rows_v.at[pl.ds(h * HC, HC)],
                            acc_sh.at[dst_v.at[j, pl.ds(h * HC, HC)]],
                            add=True)

        gather(0, 0)

        def body(j, carry):
            gwait(j, 0)
            gather(j, 1)
            scat(j, 0)
            gwait(j, 1)
            jn = lax.select(j + 1 < NCH, j + 1, 0)
            pltpu.async_copy(x_hbm.at[src_v.at[jn, pl.ds(0, HC)]],
                             rows_v.at[pl.ds(0, HC)], sem)
            scat(j, 1)
            return carry

        lax.fori_loop(0, NCH, body, 0)
        gwait(0, 0)
        plsc.subcore_barrier()
        pltpu.sync_copy(acc_sh.at[pl.ds(s * RZT, RZT)],
                        out_hbm.at[pl.ds(c * NPAD + s * RZT, RZT)])

    @functools.partial(
        pl.kernel,
        out_type=(jax.ShapeDtypeStruct((EPAD, D), jnp.float32),
                  jax.ShapeDtypeStruct((EPAD, D), jnp.float32)),
        mesh=mesh,
        scratch_types=[
            pltpu.VMEM((NCH, KC), jnp.int32),
            pltpu.VMEM((NCH, KC), jnp.int32),
            pltpu.VMEM((2, KC, D), jnp.float32),
            pltpu.VMEM((2, KC, D), jnp.float32),
            pltpu.SemaphoreType.DMA,
            pltpu.SemaphoreType.DMA,
            pltpu.SemaphoreType.DMA,
            pltpu.SemaphoreType.DMA,
        ],
    )
    def pairs_kernel(a_hbm, b_hbm, i0_hbm, i1_hbm, ga_hbm, gb_hbm,
                     i0_v, i1_v, ra_v, rb_v, ga_sem, gb_sem, wa_sem, wb_sem):
        c = lax.axis_index("c")
        s = lax.axis_index("s")
        wid = c * NS + s
        pltpu.sync_copy(i0_hbm.at[wid], i0_v)
        pltpu.sync_copy(i1_hbm.at[wid], i1_v)
        base = wid * EPTP

        def gather(j, p):
            pltpu.async_copy(a_hbm.at[i0_v.at[j]], ra_v.at[p], ga_sem)
            pltpu.async_copy(b_hbm.at[i1_v.at[j]], rb_v.at[p], gb_sem)

        def gwait(j, p):
            pltpu.make_async_copy(a_hbm.at[i0_v.at[j]], ra_v.at[p],
                                  ga_sem).wait()
            pltpu.make_async_copy(b_hbm.at[i1_v.at[j]], rb_v.at[p],
                                  gb_sem).wait()

        def write(j, p):
            dst_a = ga_hbm.at[pl.ds(base + j * KC, KC)]
            dst_b = gb_hbm.at[pl.ds(base + j * KC, KC)]
            pltpu.async_copy(ra_v.at[p], dst_a, wa_sem)
            pltpu.async_copy(rb_v.at[p], dst_b, wb_sem)

        def wwait(j, p):
            dst_a = ga_hbm.at[pl.ds(base + j * KC, KC)]
            dst_b = gb_hbm.at[pl.ds(base + j * KC, KC)]
            pltpu.make_async_copy(ra_v.at[p], dst_a, wa_sem).wait()
            pltpu.make_async_copy(rb_v.at[p], dst_b, wb_sem).wait()

        gather(0, 0)

        def body(t, carry):
            j0 = 2 * t
            gwait(j0, 0)
            gather(j0 + 1, 1)
            write(j0, 0)
            wwait(j0, 0)
            j1 = j0 + 1
            gwait(j1, 1)
            j2 = lax.select(j1 + 1 < NCH, j1 + 1, 0)
            pltpu.async_copy(a_hbm.at[i0_v.at[j2]], ra_v.at[0], ga_sem)
            pltpu.async_copy(b_hbm.at[i1_v.at[j2]], rb_v.at[0], gb_sem)
            write(j1, 1)
            wwait(j1, 1)
            return carry

        lax.fori_loop(0, NCH // 2, body, 0)
        gwait(NCH - 1, 0)
        if NCH % 2 == 1:
            write(NCH - 1, 0)
            wwait(NCH - 1, 0)

    return deg_kernel, agg_kernel, pairs_kernel


def _layer_body(with_res, parts_ref, degp_ref, x_ref, wl_ref, wr_ref, bl_ref,
                g_ref, b_ref, a_ref, out_ref):
    deg = degp_ref[0:N, 0:1] + degp_ref[NPAD:NPAD + N, 0:1]
    rd = 1.0 / jnp.clip(deg, 1.0, None)
    agg = (parts_ref[0:N, :] + parts_ref[NPAD:NPAD + N, :]) * rd
    x = x_ref[...]
    y = (jnp.dot(agg, wl_ref[...], preferred_element_type=jnp.float32)
         + bl_ref[...]
         + jnp.dot(x, wr_ref[...], preferred_element_type=jnp.float32))
    y = jax.nn.gelu(y)
    mean = jnp.mean(y, axis=0, keepdims=True)
    out = y - a_ref[...] * mean
    var = jnp.mean(out * out, axis=0, keepdims=True)
    out = out / jnp.sqrt(var + 1e-5) * g_ref[...] + b_ref[...]
    if with_res:
        out = out + x
    out_ref[...] = out


def _layer_tc(parts, degp, x, wl, wr, bl, g, b, a, with_res):
    return pl.pallas_call(
        functools.partial(_layer_body, with_res),
        out_shape=jax.ShapeDtypeStruct((N, D), jnp.float32),
    )(parts, degp, x, wl, wr, bl, g, b, a)


def _decoder_body(x1_ref, x2_ref, x3_ref, xres_ref, al_ref, w_ref, bvec_ref,
                  w1_ref, b1_ref, w2_ref, b2_ref, w3_ref, b3_ref,
                  c1_ref, cb1_ref, z_ref, a_out_ref, b_out_ref):
    al = al_ref[0, 0]
    acc = b1_ref[...]
    for k, xr in enumerate((x1_ref, x2_ref, x3_ref)):
        u = (jnp.tanh(al * xr[...]) * w_ref[:, k * D:(k + 1) * D]
             + bvec_ref[:, k * D:(k + 1) * D])
        acc = acc + jnp.dot(u, w1_ref[k * D:(k + 1) * D, :],
                            preferred_element_type=jnp.float32)
    h = jax.nn.gelu(acc)
    h = jax.nn.gelu(jnp.dot(h, w2_ref[...],
                            preferred_element_type=jnp.float32) + b2_ref[...])
    z = (jnp.dot(h, w3_ref[...], preferred_element_type=jnp.float32)
         + b3_ref[...] + xres_ref[...])
    z = z / (jnp.sqrt(jnp.sum(z * z, axis=1, keepdims=True)) + 1e-10)
    z_ref[...] = z
    a_out_ref[...] = jnp.dot(z, c1_ref[0:D, :],
                             preferred_element_type=jnp.float32)
    b_out_ref[...] = (jnp.dot(z, c1_ref[D:2 * D, :],
                              preferred_element_type=jnp.float32)
                      + cb1_ref[...])


def _decoder_tc(x1, x2, x3, xres, dyt_alpha, dyt_w, dyt_b,
                w1, b1, w2, b2, w3, b3, c1, cb1):
    return pl.pallas_call(
        _decoder_body,
        out_shape=(jax.ShapeDtypeStruct((N, D), jnp.float32),
                   jax.ShapeDtypeStruct((N, CH1), jnp.float32),
                   jax.ShapeDtypeStruct((N, CH1), jnp.float32)),
    )(x1, x2, x3, xres, dyt_alpha, dyt_w, dyt_b,
      w1, b1, w2, b2, w3, b3, c1, cb1)


BE = 4000


def _contact_body(ga_ref, gb_ref, c2_ref, cb2_ref, c3_ref, cb3_ref, out_ref):
    h = jax.nn.gelu(ga_ref[...] + gb_ref[...])
    h = jax.nn.gelu(jnp.dot(h, c2_ref[...],
                            preferred_element_type=jnp.float32) + cb2_ref[...])
    logit = jnp.sum(h * c3_ref[...], axis=1, keepdims=True) + cb3_ref[...]
    out_ref[...] = jax.nn.sigmoid(logit)


def _contact_tc(ga, gb, c2, cb2, c3row, cb3):
    return pl.pallas_call(
        _contact_body,
        grid=(E // BE,),
        in_specs=[
            pl.BlockSpec((BE, D), lambda i: (i, 0)),
            pl.BlockSpec((BE, D), lambda i: (i, 0)),
            pl.BlockSpec((D, CH2), lambda i: (0, 0)),
            pl.BlockSpec((1, CH2), lambda i: (0, 0)),
            pl.BlockSpec((1, CH2), lambda i: (0, 0)),
            pl.BlockSpec((1, 1), lambda i: (0, 0)),
        ],
        out_specs=pl.BlockSpec((BE, 1), lambda i: (i, 0)),
        out_shape=jax.ShapeDtypeStruct((E, 1), jnp.float32),
    )(ga, gb, c2, cb2, c3row, cb3)


def kernel(x_res, edge_index, contact_pred_index, Wl, Wr, bl, gn_gamma,
           gn_beta, gn_alpha, dyt_alpha, dyt_w, dyt_b, W1, b1, W2, b2, W3,
           b3, C1, cb1, C2, cb2, C3, cb3):
    def _pad(ix, ext):
        return jnp.concatenate([ix.astype(jnp.int32), ext]
                               ).reshape(NW, NCH, KC)

    spread = jnp.arange(EPAD - E, dtype=jnp.int32)
    src = _pad(edge_index[0], spread % N)
    dst = _pad(edge_index[1], N + spread % (NPAD - N))
    i0 = _pad(contact_pred_index[0], spread % N)
    i1 = _pad(contact_pred_index[1], (spread * 7 + 3) % N)

    zrows = jnp.zeros((RZT, D), jnp.float32)

    deg_kernel, agg_kernel, pairs_kernel = _sc_kernels()
    ones_kc = jnp.ones((KC, DW), jnp.float32)
    degp = deg_kernel(dst, zrows, ones_kc)

    x = x_res
    feats = []
    for i in range(L):
        parts = agg_kernel(x, src, dst, zrows)
        x = _layer_tc(parts, degp, x, Wl[i], Wr[i],
                      bl[i].reshape(1, D), gn_gamma[i].reshape(1, D),
                      gn_beta[i].reshape(1, D), gn_alpha[i].reshape(1, D),
                      with_res=(i > 0))
        feats.append(x)

    z, a_tab, b_tab = _decoder_tc(
        feats[0], feats[1], feats[2], x_res,
        dyt_alpha.reshape(1, 1), dyt_w.reshape(1, L * D),
        dyt_b.reshape(1, L * D), W1, b1.reshape(1, XH), W2,
        b2.reshape(1, XH), W3, b3.reshape(1, D), C1, cb1.reshape(1, CH1))

    ga, gb = pairs_kernel(a_tab, b_tab, i0, i1)

    contact = _contact_tc(ga, gb, C2, cb2.reshape(1, CH2),
                          C3.reshape(1, CH2), cb3.reshape(1, 1))
    return z, contact


# speedup vs baseline: 1.0368x; 1.0368x over previous
import functools

import jax
import jax.numpy as jnp
from jax import lax
from jax.experimental import pallas as pl
from jax.experimental.pallas import tpu as pltpu
from jax.experimental.pallas import tpu_sc as plsc

N = 10000
E = 320000
D = 128
L = 3
XH = 128
CH1 = 128
CH2 = 64

NC = 2
NS = 16
NW = NC * NS
KC = 128
NCH = 79
EPTP = NCH * KC
EPAD = NW * EPTP
HC = KC // 2
RPT = N // NS
NPAD = 10240
RZT = NPAD // NS
DW = 128


@functools.cache
def _sc_kernels():
    mesh = plsc.VectorSubcoreMesh(core_axis_name="c", subcore_axis_name="s",
                                  num_cores=NC, num_subcores=NS)

    @functools.partial(
        pl.kernel,
        out_type=jax.ShapeDtypeStruct((NC * NPAD, DW), jnp.float32),
        mesh=mesh,
        scratch_types=[
            pltpu.VMEM((NCH, KC), jnp.int32),
            pltpu.VMEM((KC, DW), jnp.float32),
            pltpu.VMEM_SHARED((NPAD, DW), jnp.float32),
        ],
    )
    def deg_kernel(dst_hbm, zeros_hbm, ones_hbm, out_hbm, dst_v, ones_v,
                   deg_sh):
        c = lax.axis_index("c")
        s = lax.axis_index("s")
        wid = c * NS + s
        pltpu.sync_copy(zeros_hbm, deg_sh.at[pl.ds(s * RZT, RZT)])
        pltpu.sync_copy(dst_hbm.at[wid], dst_v)
        pltpu.sync_copy(ones_hbm, ones_v)
        plsc.subcore_barrier()

        def body(j, carry):
            pltpu.sync_copy(ones_v, deg_sh.at[dst_v.at[j]], add=True)
            return carry

        lax.fori_loop(0, NCH, body, 0)
        plsc.subcore_barrier()
        pltpu.sync_copy(deg_sh.at[pl.ds(s * RZT, RZT)],
                        out_hbm.at[pl.ds(c * NPAD + s * RZT, RZT)])

    @functools.partial(
        pl.kernel,
        out_type=jax.ShapeDtypeStruct((NC * NPAD, D), jnp.float32),
        mesh=mesh,
        scratch_types=[
            pltpu.VMEM((NCH, KC), jnp.int32),
            pltpu.VMEM((NCH, KC), jnp.int32),
            pltpu.VMEM((KC, D), jnp.float32),
            pltpu.VMEM_SHARED((NPAD, D), jnp.float32),
            pltpu.SemaphoreType.DMA,
        ],
    )
    def agg_kernel(x_hbm, src_hbm, dst_hbm, zrows_hbm, out_hbm,
                   src_v, dst_v, rows_v, acc_sh, sem):
        c = lax.axis_index("c")
        s = lax.axis_index("s")
        wid = c * NS + s
        pltpu.sync_copy(zrows_hbm, acc_sh.at[pl.ds(s * RZT, RZT)])
        pltpu.sync_copy(src_hbm.at[wid], src_v)
        pltpu.sync_copy(dst_hbm.at[wid], dst_v)
        plsc.subcore_barrier()

        def gather(j, h):
            pltpu.async_copy(x_hbm.at[src_v.at[j, pl.ds(h * HC, HC)]],
                             rows_v.at[pl.ds(h * HC, HC)], sem)

        def gwait(j, h):
            pltpu.make_async_copy(x_hbm.at[src_v.at[j, pl.ds(h * HC, HC)]],
                                  rows_v.at[pl.ds(h * HC, HC)], sem).wait()

        def scat(j, h):
            pltpu.sync_copy(rows_v.at[pl.ds(h * HC, HC)],
                            acc_sh.at[dst_v.at[j, pl.ds(h * HC, HC)]],
                            add=True)

        gather(0, 0)

        def body(j, carry):
            gwait(j, 0)
            gather(j, 1)
            scat(j, 0)
            gwait(j, 1)
            jn = lax.select(j + 1 < NCH, j + 1, 0)
            pltpu.async_copy(x_hbm.at[src_v.at[jn, pl.ds(0, HC)]],
                             rows_v.at[pl.ds(0, HC)], sem)
            scat(j, 1)
            return carry

        lax.fori_loop(0, NCH, body, 0)
        gwait(0, 0)
        plsc.subcore_barrier()
        pltpu.sync_copy(acc_sh.at[pl.ds(s * RZT, RZT)],
                        out_hbm.at[pl.ds(c * NPAD + s * RZT, RZT)])

    @functools.partial(
        pl.kernel,
        out_type=jax.ShapeDtypeStruct((EPAD, D), jnp.float32),
        mesh=mesh,
        scratch_types=[
            pltpu.VMEM((NCH, KC), jnp.int32),
            pltpu.VMEM((NCH, KC), jnp.int32),
            pltpu.VMEM((2, KC, D), jnp.float32),
            pltpu.SemaphoreType.DMA,
            pltpu.SemaphoreType.DMA,
        ],
    )
    def pairs_kernel(a_hbm, b_hbm, i0_hbm, i1_hbm, g_hbm,
                     i0_v, i1_v, r_v, g_sem, w_sem):
        c = lax.axis_index("c")
        s = lax.axis_index("s")
        wid = c * NS + s
        pltpu.sync_copy(i0_hbm.at[wid], i0_v)
        pltpu.sync_copy(i1_hbm.at[wid], i1_v)
        base = wid * EPTP

        def gather_a(j, p):
            pltpu.async_copy(a_hbm.at[i0_v.at[j]], r_v.at[p], g_sem)

        def gwait_a(j, p):
            pltpu.make_async_copy(a_hbm.at[i0_v.at[j]], r_v.at[p],
                                  g_sem).wait()

        def add_b(j, p):
            pltpu.sync_copy(b_hbm.at[i1_v.at[j]], r_v.at[p], add=True)

        def write(j, p):
            pltpu.async_copy(r_v.at[p], g_hbm.at[pl.ds(base + j * KC, KC)],
                             w_sem)

        def wwait(j, p):
            pltpu.make_async_copy(r_v.at[p],
                                  g_hbm.at[pl.ds(base + j * KC, KC)],
                                  w_sem).wait()

        gather_a(0, 0)

        def body(t, carry):
            j0 = 2 * t
            gwait_a(j0, 0)
            gather_a(j0 + 1, 1)
            add_b(j0, 0)
            write(j0, 0)
            wwait(j0, 0)
            j1 = j0 + 1
            gwait_a(j1, 1)
            j2 = lax.select(j1 + 1 < NCH, j1 + 1, 0)
            pltpu.async_copy(a_hbm.at[i0_v.at[j2]], r_v.at[0], g_sem)
            add_b(j1, 1)
            write(j1, 1)
            wwait(j1, 1)
            return carry

        lax.fori_loop(0, NCH // 2, body, 0)
        gwait_a(NCH - 1, 0)
        if NCH % 2 == 1:
            add_b(NCH - 1, 0)
            write(NCH - 1, 0)
            wwait(NCH - 1, 0)

    return deg_kernel, agg_kernel, pairs_kernel


def _layer_body(with_res, parts_ref, degp_ref, x_ref, wl_ref, wr_ref, bl_ref,
                g_ref, b_ref, a_ref, out_ref):
    deg = degp_ref[0:N, 0:1] + degp_ref[NPAD:NPAD + N, 0:1]
    rd = 1.0 / jnp.clip(deg, 1.0, None)
    agg = (parts_ref[0:N, :] + parts_ref[NPAD:NPAD + N, :]) * rd
    x = x_ref[...]
    y = (jnp.dot(agg, wl_ref[...], preferred_element_type=jnp.float32)
         + bl_ref[...]
         + jnp.dot(x, wr_ref[...], preferred_element_type=jnp.float32))
    y = jax.nn.gelu(y)
    mean = jnp.mean(y, axis=0, keepdims=True)
    out = y - a_ref[...] * mean
    var = jnp.mean(out * out, axis=0, keepdims=True)
    out = out / jnp.sqrt(var + 1e-5) * g_ref[...] + b_ref[...]
    if with_res:
        out = out + x
    out_ref[...] = out


def _layer_tc(parts, degp, x, wl, wr, bl, g, b, a, with_res):
    return pl.pallas_call(
        functools.partial(_layer_body, with_res),
        out_shape=jax.ShapeDtypeStruct((N, D), jnp.float32),
    )(parts, degp, x, wl, wr, bl, g, b, a)


def _decoder_body(x1_ref, x2_ref, x3_ref, xres_ref, al_ref, w_ref, bvec_ref,
                  w1_ref, b1_ref, w2_ref, b2_ref, w3_ref, b3_ref,
                  c1_ref, cb1_ref, z_ref, a_out_ref, b_out_ref):
    al = al_ref[0, 0]
    acc = b1_ref[...]
    for k, xr in enumerate((x1_ref, x2_ref, x3_ref)):
        u = (jnp.tanh(al * xr[...]) * w_ref[:, k * D:(k + 1) * D]
             + bvec_ref[:, k * D:(k + 1) * D])
        acc = acc + jnp.dot(u, w1_ref[k * D:(k + 1) * D, :],
                            preferred_element_type=jnp.float32)
    h = jax.nn.gelu(acc)
    h = jax.nn.gelu(jnp.dot(h, w2_ref[...],
                            preferred_element_type=jnp.float32) + b2_ref[...])
    z = (jnp.dot(h, w3_ref[...], preferred_element_type=jnp.float32)
         + b3_ref[...] + xres_ref[...])
    z = z / (jnp.sqrt(jnp.sum(z * z, axis=1, keepdims=True)) + 1e-10)
    z_ref[...] = z
    a_out_ref[...] = jnp.dot(z, c1_ref[0:D, :],
                             preferred_element_type=jnp.float32)
    b_out_ref[...] = (jnp.dot(z, c1_ref[D:2 * D, :],
                              preferred_element_type=jnp.float32)
                      + cb1_ref[...])


def _decoder_tc(x1, x2, x3, xres, dyt_alpha, dyt_w, dyt_b,
                w1, b1, w2, b2, w3, b3, c1, cb1):
    return pl.pallas_call(
        _decoder_body,
        out_shape=(jax.ShapeDtypeStruct((N, D), jnp.float32),
                   jax.ShapeDtypeStruct((N, CH1), jnp.float32),
                   jax.ShapeDtypeStruct((N, CH1), jnp.float32)),
    )(x1, x2, x3, xres, dyt_alpha, dyt_w, dyt_b,
      w1, b1, w2, b2, w3, b3, c1, cb1)


BE = 4000


def _contact_body(g_ref, c2_ref, cb2_ref, c3_ref, cb3_ref, out_ref):
    h = jax.nn.gelu(g_ref[...])
    h = jax.nn.gelu(jnp.dot(h, c2_ref[...],
                            preferred_element_type=jnp.float32) + cb2_ref[...])
    logit = jnp.sum(h * c3_ref[...], axis=1, keepdims=True) + cb3_ref[...]
    out_ref[...] = jax.nn.sigmoid(logit)


def _contact_tc(g, c2, cb2, c3row, cb3):
    return pl.pallas_call(
        _contact_body,
        grid=(E // BE,),
        in_specs=[
            pl.BlockSpec((BE, D), lambda i: (i, 0)),
            pl.BlockSpec((D, CH2), lambda i: (0, 0)),
            pl.BlockSpec((1, CH2), lambda i: (0, 0)),
            pl.BlockSpec((1, CH2), lambda i: (0, 0)),
            pl.BlockSpec((1, 1), lambda i: (0, 0)),
        ],
        out_specs=pl.BlockSpec((BE, 1), lambda i: (i, 0)),
        out_shape=jax.ShapeDtypeStruct((E, 1), jnp.float32),
    )(g, c2, cb2, c3row, cb3)


def kernel(x_res, edge_index, contact_pred_index, Wl, Wr, bl, gn_gamma,
           gn_beta, gn_alpha, dyt_alpha, dyt_w, dyt_b, W1, b1, W2, b2, W3,
           b3, C1, cb1, C2, cb2, C3, cb3):
    def _pad(ix, ext):
        return jnp.concatenate([ix.astype(jnp.int32), ext]
                               ).reshape(NW, NCH, KC)

    spread = jnp.arange(EPAD - E, dtype=jnp.int32)
    src = _pad(edge_index[0], spread % N)
    dst = _pad(edge_index[1], N + spread % (NPAD - N))
    i0 = _pad(contact_pred_index[0], spread % N)
    i1 = _pad(contact_pred_index[1], (spread * 7 + 3) % N)

    zrows = jnp.zeros((RZT, D), jnp.float32)

    deg_kernel, agg_kernel, pairs_kernel = _sc_kernels()
    ones_kc = jnp.ones((KC, DW), jnp.float32)
    degp = deg_kernel(dst, zrows, ones_kc)

    x = x_res
    feats = []
    for i in range(L):
        parts = agg_kernel(x, src, dst, zrows)
        x = _layer_tc(parts, degp, x, Wl[i], Wr[i],
                      bl[i].reshape(1, D), gn_gamma[i].reshape(1, D),
                      gn_beta[i].reshape(1, D), gn_alpha[i].reshape(1, D),
                      with_res=(i > 0))
        feats.append(x)

    z, a_tab, b_tab = _decoder_tc(
        feats[0], feats[1], feats[2], x_res,
        dyt_alpha.reshape(1, 1), dyt_w.reshape(1, L * D),
        dyt_b.reshape(1, L * D), W1, b1.reshape(1, XH), W2,
        b2.reshape(1, XH), W3, b3.reshape(1, D), C1, cb1.reshape(1, CH1))

    g = pairs_kernel(a_tab, b_tab, i0, i1)

    contact = _contact_tc(g, C2, cb2.reshape(1, CH2),
                          C3.reshape(1, CH2), cb3.reshape(1, 1))
    return z, contact
